# baseline (speedup 1.0000x reference)
import jax
import jax.numpy as jnp
from jax import lax
from jax.experimental import pallas as pl
from jax.experimental.pallas import tpu as pltpu
from jax.experimental.pallas import tpu_sc as plsc

_BATCH = 1024
_EMB = 64
_VOCAB = 100000

_NUM_WORKERS = 32
_ROWS_PER_WORKER = _BATCH // _NUM_WORKERS

_BATCH_TILE = 16


def _sc_gather_pairs(table2, idx_half):
    mesh = plsc.VectorSubcoreMesh(core_axis_name="c", subcore_axis_name="s")

    @pl.kernel(
        mesh=mesh,
        out_type=jax.ShapeDtypeStruct((_BATCH, 2 * _EMB), table2.dtype),
        scratch_types=[
            pltpu.VMEM((_ROWS_PER_WORKER,), jnp.int32),
            pltpu.VMEM((_ROWS_PER_WORKER, 2 * _EMB), table2.dtype),
            pltpu.SemaphoreType.DMA,
        ],
    )
    def gather_kernel(table_hbm, idx_hbm, out_hbm, idx_v, rows_v, sem):
        wid = lax.axis_index("s") * 2 + lax.axis_index("c")
        base = wid * _ROWS_PER_WORKER
        pltpu.sync_copy(idx_hbm.at[pl.ds(base, _ROWS_PER_WORKER)], idx_v)
        pltpu.async_copy(table_hbm.at[idx_v], rows_v, sem).wait()
        pltpu.sync_copy(rows_v, out_hbm.at[pl.ds(base, _ROWS_PER_WORKER)])

    return gather_kernel(table2, idx_half)


def _fused_body(e2_ref, p_ref, w_ref, b_ref, o_ref):
    e2 = e2_ref[...]
    par = p_ref[...] == 1
    e = jnp.where(par, e2[:, _EMB:], e2[:, :_EMB]).astype(jnp.bfloat16)
    x = lax.dot_general(
        e, w_ref[...], (((1,), (0,)), ((), ())),
        preferred_element_type=jnp.float32,
    )
    x = x + b_ref[...]
    m = jnp.max(x, axis=1, keepdims=True)
    lse = jnp.log(jnp.sum(jnp.exp(x - m), axis=1, keepdims=True)) + m
    o_ref[...] = x - lse


def _tc_logsoftmax(embeds2, parity, w_t, b2d):
    return pl.pallas_call(
        _fused_body,
        grid=(_BATCH // _BATCH_TILE,),
        in_specs=[
            pl.BlockSpec((_BATCH_TILE, 2 * _EMB), lambda i: (i, 0)),
            pl.BlockSpec((_BATCH_TILE, 1), lambda i: (i, 0)),
            pl.BlockSpec((_EMB, _VOCAB), lambda i: (0, 0)),
            pl.BlockSpec((1, _VOCAB), lambda i: (0, 0)),
        ],
        out_specs=pl.BlockSpec((_BATCH_TILE, _VOCAB), lambda i: (i, 0)),
        out_shape=jax.ShapeDtypeStruct((_BATCH, _VOCAB), jnp.float32),
    )(embeds2, parity, w_t, b2d)


def kernel(input_word, emb_table, W, b):
    idx = input_word.astype(jnp.int32)
    table2 = emb_table.reshape(_VOCAB // 2, 2 * _EMB)
    embeds2 = _sc_gather_pairs(table2, idx >> 1)
    parity = (idx & 1).reshape(_BATCH, 1)
    w_t = W.T.astype(jnp.bfloat16)
    return _tc_logsoftmax(embeds2, parity, w_t, b.reshape(1, _VOCAB))
